# Initial kernel scaffold; baseline (speedup 1.0000x reference)
import jax
import jax.numpy as jnp
from jax.experimental import pallas as pl


def kernel(x, edge_index, W1, b1, g1, beta1, Wl, bl, Wr, g2, beta2):
    raise NotImplementedError("write your pallas kernel here")


# speedup vs baseline: 6.1399x; 6.1399x over previous
import functools

import jax
import jax.numpy as jnp
from jax import lax
from jax.experimental import pallas as pl
from jax.experimental.pallas import tpu as pltpu
from jax.experimental.pallas import tpu_sc as plsc

N, E, D, H = 10000, 320000, 128, 128
DE = 144
K = 80
NCORES = 2
NSUB = 16
NTILES = NCORES * NSUB
EPT = E // NTILES
CH = EPT // K
RPS = N // NSUB
BM = 2000
NB = N // BM
EPS = 1e-5


def _lin_stats_body(x_ref, w_ref, b_ref, h_ref, st_ref, acc_ref):
    i = pl.program_id(0)

    @pl.when(i == 0)
    def _():
        acc_ref[...] = jnp.zeros_like(acc_ref)

    h = jnp.dot(x_ref[...], w_ref[...], preferred_element_type=jnp.float32)
    h = h + b_ref[...]
    h_ref[...] = h
    acc_ref[0:1, :] += jnp.sum(h, axis=0, keepdims=True)
    acc_ref[1:2, :] += jnp.sum(h * h, axis=0, keepdims=True)

    @pl.when(i == NB - 1)
    def _():
        st_ref[...] = acc_ref[...]


def _bn_relu_body(h_ref, st_ref, g_ref, be_ref, f_ref, fx_ref):
    m = st_ref[0:1, :] / N
    v = st_ref[1:2, :] / N - m * m
    inv = g_ref[...] * lax.rsqrt(v + EPS)
    f = jnp.maximum((h_ref[...] - m) * inv + be_ref[...], 0.0)
    f_ref[...] = f
    ones = jnp.ones((BM, 1), jnp.float32)
    pad = jnp.zeros((BM, DE - D - 1), jnp.float32)
    fx_ref[...] = jnp.concatenate([f, ones, pad], axis=1)


def _sc_agg_body(fx_hbm, src_hbm, dst_hbm, zeros_hbm, out_hbm,
                 src_v, dst_v, rows_v, agg_sh, sem):
    c = lax.axis_index("c")
    s = lax.axis_index("s")
    wid = c * NSUB + s
    pltpu.sync_copy(zeros_hbm, agg_sh.at[pl.ds(s * RPS, RPS)])
    pltpu.sync_copy(src_hbm.at[wid], src_v)
    pltpu.sync_copy(dst_hbm.at[wid], dst_v)
    plsc.subcore_barrier()

    def body(j, carry):
        pltpu.async_copy(fx_hbm.at[src_v.at[j]], rows_v, sem).wait()
        pltpu.sync_copy(rows_v, agg_sh.at[dst_v.at[j]], add=True)
        return carry

    lax.fori_loop(0, CH, body, 0)
    plsc.subcore_barrier()
    pltpu.sync_copy(agg_sh.at[pl.ds(s * RPS, RPS)], out_hbm.at[wid])


def _make_sc_agg():
    mesh = plsc.VectorSubcoreMesh(core_axis_name="c", subcore_axis_name="s")
    return pl.kernel(
        _sc_agg_body,
        out_type=jax.ShapeDtypeStruct((NTILES, RPS, DE), jnp.float32),
        mesh=mesh,
        scratch_types=[
            pltpu.VMEM((CH, K), jnp.int32),
            pltpu.VMEM((CH, K), jnp.int32),
            pltpu.VMEM((K, DE), jnp.float32),
            pltpu.VMEM_SHARED((N, DE), jnp.float32),
            pltpu.SemaphoreType.DMA,
        ],
        compiler_params=pltpu.CompilerParams(use_tc_tiling_on_sc=False),
    )


def _conv_stats_body(a0_ref, a1_ref, f_ref, wl_ref, wr_ref, bl_ref,
                     o_ref, st_ref, acc_ref):
    i = pl.program_id(0)

    @pl.when(i == 0)
    def _():
        acc_ref[...] = jnp.zeros_like(acc_ref)

    a = a0_ref[...] + a1_ref[...]
    deg = jnp.maximum(a[:, D:D + 1], 1.0)
    mean = a[:, :D] / deg
    o = jnp.dot(mean, wl_ref[...], preferred_element_type=jnp.float32)
    o = o + jnp.dot(f_ref[...], wr_ref[...], preferred_element_type=jnp.float32)
    o = o + bl_ref[...]
    o_ref[...] = o
    acc_ref[0:1, :] += jnp.sum(o, axis=0, keepdims=True)
    acc_ref[1:2, :] += jnp.sum(o * o, axis=0, keepdims=True)

    @pl.when(i == NB - 1)
    def _():
        st_ref[...] = acc_ref[...]


def _bn_body(o_ref, st_ref, g_ref, be_ref, out_ref):
    m = st_ref[0:1, :] / N
    v = st_ref[1:2, :] / N - m * m
    inv = g_ref[...] * lax.rsqrt(v + EPS)
    out_ref[...] = (o_ref[...] - m) * inv + be_ref[...]


def kernel(x, edge_index, W1, b1, g1, beta1, Wl, bl, Wr, g2, beta2):
    src = edge_index[0].reshape(NTILES, CH, K)
    dst = edge_index[1].reshape(NTILES, CH, K)

    h, st1 = pl.pallas_call(
        _lin_stats_body,
        grid=(NB,),
        in_specs=[
            pl.BlockSpec((BM, D), lambda i: (i, 0)),
            pl.BlockSpec((D, H), lambda i: (0, 0)),
            pl.BlockSpec((1, H), lambda i: (0, 0)),
        ],
        out_specs=[
            pl.BlockSpec((BM, H), lambda i: (i, 0)),
            pl.BlockSpec((8, H), lambda i: (0, 0)),
        ],
        out_shape=[
            jax.ShapeDtypeStruct((N, H), jnp.float32),
            jax.ShapeDtypeStruct((8, H), jnp.float32),
        ],
        scratch_shapes=[pltpu.VMEM((8, H), jnp.float32)],
    )(x, W1, b1.reshape(1, H))

    feat, fx = pl.pallas_call(
        _bn_relu_body,
        grid=(NB,),
        in_specs=[
            pl.BlockSpec((BM, H), lambda i: (i, 0)),
            pl.BlockSpec((8, H), lambda i: (0, 0)),
            pl.BlockSpec((1, H), lambda i: (0, 0)),
            pl.BlockSpec((1, H), lambda i: (0, 0)),
        ],
        out_specs=[
            pl.BlockSpec((BM, H), lambda i: (i, 0)),
            pl.BlockSpec((BM, DE), lambda i: (i, 0)),
        ],
        out_shape=[
            jax.ShapeDtypeStruct((N, H), jnp.float32),
            jax.ShapeDtypeStruct((N, DE), jnp.float32),
        ],
    )(h, st1, g1.reshape(1, H), beta1.reshape(1, H))

    zeros = jnp.zeros((RPS, DE), jnp.float32)
    aggp = _make_sc_agg()(fx, src, dst, zeros)
    aggp = aggp.reshape(NCORES, N, DE)

    out_raw, st2 = pl.pallas_call(
        _conv_stats_body,
        grid=(NB,),
        in_specs=[
            pl.BlockSpec((BM, DE), lambda i: (i, 0)),
            pl.BlockSpec((BM, DE), lambda i: (i, 0)),
            pl.BlockSpec((BM, H), lambda i: (i, 0)),
            pl.BlockSpec((H, H), lambda i: (0, 0)),
            pl.BlockSpec((H, H), lambda i: (0, 0)),
            pl.BlockSpec((1, H), lambda i: (0, 0)),
        ],
        out_specs=[
            pl.BlockSpec((BM, H), lambda i: (i, 0)),
            pl.BlockSpec((8, H), lambda i: (0, 0)),
        ],
        out_shape=[
            jax.ShapeDtypeStruct((N, H), jnp.float32),
            jax.ShapeDtypeStruct((8, H), jnp.float32),
        ],
        scratch_shapes=[pltpu.VMEM((8, H), jnp.float32)],
    )(aggp[0], aggp[1], feat, Wl, Wr, bl.reshape(1, H))

    out_feat = pl.pallas_call(
        _bn_body,
        grid=(NB,),
        in_specs=[
            pl.BlockSpec((BM, H), lambda i: (i, 0)),
            pl.BlockSpec((8, H), lambda i: (0, 0)),
            pl.BlockSpec((1, H), lambda i: (0, 0)),
            pl.BlockSpec((1, H), lambda i: (0, 0)),
        ],
        out_specs=pl.BlockSpec((BM, H), lambda i: (i, 0)),
        out_shape=jax.ShapeDtypeStruct((N, H), jnp.float32),
    )(out_raw, st2, g2.reshape(1, H), beta2.reshape(1, H))

    return (feat, out_feat)


# speedup vs baseline: 6.9615x; 1.1338x over previous
import functools

import jax
import jax.numpy as jnp
from jax import lax
from jax.experimental import pallas as pl
from jax.experimental.pallas import tpu as pltpu
from jax.experimental.pallas import tpu_sc as plsc

N, E, D, H = 10000, 320000, 128, 128
DE = 144
K = 50
NCORES = 2
NSUB = 16
NTILES = NCORES * NSUB
EPT = E // NTILES
CH = EPT // K
RPS = N // NSUB
BM = 2000
NB = N // BM
EPS = 1e-5


def _lin_stats_body(x_ref, w_ref, b_ref, h_ref, st_ref, acc_ref):
    i = pl.program_id(0)

    @pl.when(i == 0)
    def _():
        acc_ref[...] = jnp.zeros_like(acc_ref)

    h = jnp.dot(x_ref[...], w_ref[...], preferred_element_type=jnp.float32)
    h = h + b_ref[...]
    h_ref[...] = h
    acc_ref[0:1, :] += jnp.sum(h, axis=0, keepdims=True)
    acc_ref[1:2, :] += jnp.sum(h * h, axis=0, keepdims=True)

    @pl.when(i == NB - 1)
    def _():
        st_ref[...] = acc_ref[...]


def _bn_relu_body(h_ref, st_ref, g_ref, be_ref, f_ref, fx_ref):
    m = st_ref[0:1, :] / N
    v = st_ref[1:2, :] / N - m * m
    inv = g_ref[...] * lax.rsqrt(v + EPS)
    f = jnp.maximum((h_ref[...] - m) * inv + be_ref[...], 0.0)
    f_ref[...] = f
    ones = jnp.ones((BM, 1), jnp.float32)
    pad = jnp.zeros((BM, DE - D - 1), jnp.float32)
    fx_ref[...] = jnp.concatenate([f, ones, pad], axis=1)


def _sc_agg_body(fx_hbm, src_hbm, dst_hbm, zeros_hbm, out_hbm,
                 src_v, dst_v, rows0_v, rows1_v, agg_sh, sem0, sem1):
    c = lax.axis_index("c")
    s = lax.axis_index("s")
    wid = c * NSUB + s
    pltpu.sync_copy(src_hbm.at[wid], src_v)
    pltpu.sync_copy(dst_hbm.at[wid], dst_v)
    pltpu.sync_copy(zeros_hbm, agg_sh.at[pl.ds(s * RPS, RPS)])
    plsc.subcore_barrier()

    pltpu.async_copy(fx_hbm.at[src_v.at[0]], rows0_v, sem0)

    def body(t, carry):
        j0 = 2 * t
        pltpu.async_copy(fx_hbm.at[src_v.at[j0 + 1]], rows1_v, sem1)
        pltpu.make_async_copy(fx_hbm.at[src_v.at[j0]], rows0_v, sem0).wait()
        pltpu.sync_copy(rows0_v, agg_sh.at[dst_v.at[j0]], add=True)

        @pl.when(j0 + 2 < CH)
        def _():
            pltpu.async_copy(fx_hbm.at[src_v.at[j0 + 2]], rows0_v, sem0)

        pltpu.make_async_copy(fx_hbm.at[src_v.at[j0 + 1]], rows1_v, sem1).wait()
        pltpu.sync_copy(rows1_v, agg_sh.at[dst_v.at[j0 + 1]], add=True)
        return carry

    lax.fori_loop(0, CH // 2, body, 0)
    plsc.subcore_barrier()
    pltpu.sync_copy(agg_sh.at[pl.ds(s * RPS, RPS)], out_hbm.at[wid])


def _make_sc_agg():
    mesh = plsc.VectorSubcoreMesh(core_axis_name="c", subcore_axis_name="s")
    return pl.kernel(
        _sc_agg_body,
        out_type=jax.ShapeDtypeStruct((NTILES, RPS, DE), jnp.float32),
        mesh=mesh,
        scratch_types=[
            pltpu.VMEM((CH, K), jnp.int32),
            pltpu.VMEM((CH, K), jnp.int32),
            pltpu.VMEM((K, DE), jnp.float32),
            pltpu.VMEM((K, DE), jnp.float32),
            pltpu.VMEM_SHARED((N, DE), jnp.float32),
            pltpu.SemaphoreType.DMA,
            pltpu.SemaphoreType.DMA,
        ],
        compiler_params=pltpu.CompilerParams(use_tc_tiling_on_sc=False),
    )


def _conv_stats_body(a0_ref, a1_ref, f_ref, wl_ref, wr_ref, bl_ref,
                     o_ref, st_ref, acc_ref):
    i = pl.program_id(0)

    @pl.when(i == 0)
    def _():
        acc_ref[...] = jnp.zeros_like(acc_ref)

    a = a0_ref[...] + a1_ref[...]
    deg = jnp.maximum(a[:, D:D + 1], 1.0)
    mean = a[:, :D] / deg
    o = jnp.dot(mean, wl_ref[...], preferred_element_type=jnp.float32)
    o = o + jnp.dot(f_ref[...], wr_ref[...], preferred_element_type=jnp.float32)
    o = o + bl_ref[...]
    o_ref[...] = o
    acc_ref[0:1, :] += jnp.sum(o, axis=0, keepdims=True)
    acc_ref[1:2, :] += jnp.sum(o * o, axis=0, keepdims=True)

    @pl.when(i == NB - 1)
    def _():
        st_ref[...] = acc_ref[...]


def _bn_body(o_ref, st_ref, g_ref, be_ref, out_ref):
    m = st_ref[0:1, :] / N
    v = st_ref[1:2, :] / N - m * m
    inv = g_ref[...] * lax.rsqrt(v + EPS)
    out_ref[...] = (o_ref[...] - m) * inv + be_ref[...]


def kernel(x, edge_index, W1, b1, g1, beta1, Wl, bl, Wr, g2, beta2):
    src = edge_index[0].reshape(NTILES, CH, K)
    dst = edge_index[1].reshape(NTILES, CH, K)

    h, st1 = pl.pallas_call(
        _lin_stats_body,
        grid=(NB,),
        in_specs=[
            pl.BlockSpec((BM, D), lambda i: (i, 0)),
            pl.BlockSpec((D, H), lambda i: (0, 0)),
            pl.BlockSpec((1, H), lambda i: (0, 0)),
        ],
        out_specs=[
            pl.BlockSpec((BM, H), lambda i: (i, 0)),
            pl.BlockSpec((8, H), lambda i: (0, 0)),
        ],
        out_shape=[
            jax.ShapeDtypeStruct((N, H), jnp.float32),
            jax.ShapeDtypeStruct((8, H), jnp.float32),
        ],
        scratch_shapes=[pltpu.VMEM((8, H), jnp.float32)],
    )(x, W1, b1.reshape(1, H))

    feat, fx = pl.pallas_call(
        _bn_relu_body,
        grid=(NB,),
        in_specs=[
            pl.BlockSpec((BM, H), lambda i: (i, 0)),
            pl.BlockSpec((8, H), lambda i: (0, 0)),
            pl.BlockSpec((1, H), lambda i: (0, 0)),
            pl.BlockSpec((1, H), lambda i: (0, 0)),
        ],
        out_specs=[
            pl.BlockSpec((BM, H), lambda i: (i, 0)),
            pl.BlockSpec((BM, DE), lambda i: (i, 0)),
        ],
        out_shape=[
            jax.ShapeDtypeStruct((N, H), jnp.float32),
            jax.ShapeDtypeStruct((N, DE), jnp.float32),
        ],
    )(h, st1, g1.reshape(1, H), beta1.reshape(1, H))

    zeros = jnp.zeros((RPS, DE), jnp.float32)
    aggp = _make_sc_agg()(fx, src, dst, zeros)
    aggp = aggp.reshape(NCORES, N, DE)

    out_raw, st2 = pl.pallas_call(
        _conv_stats_body,
        grid=(NB,),
        in_specs=[
            pl.BlockSpec((BM, DE), lambda i: (i, 0)),
            pl.BlockSpec((BM, DE), lambda i: (i, 0)),
            pl.BlockSpec((BM, H), lambda i: (i, 0)),
            pl.BlockSpec((H, H), lambda i: (0, 0)),
            pl.BlockSpec((H, H), lambda i: (0, 0)),
            pl.BlockSpec((1, H), lambda i: (0, 0)),
        ],
        out_specs=[
            pl.BlockSpec((BM, H), lambda i: (i, 0)),
            pl.BlockSpec((8, H), lambda i: (0, 0)),
        ],
        out_shape=[
            jax.ShapeDtypeStruct((N, H), jnp.float32),
            jax.ShapeDtypeStruct((8, H), jnp.float32),
        ],
        scratch_shapes=[pltpu.VMEM((8, H), jnp.float32)],
    )(aggp[0], aggp[1], feat, Wl, Wr, bl.reshape(1, H))

    out_feat = pl.pallas_call(
        _bn_body,
        grid=(NB,),
        in_specs=[
            pl.BlockSpec((BM, H), lambda i: (i, 0)),
            pl.BlockSpec((8, H), lambda i: (0, 0)),
            pl.BlockSpec((1, H), lambda i: (0, 0)),
            pl.BlockSpec((1, H), lambda i: (0, 0)),
        ],
        out_specs=pl.BlockSpec((BM, H), lambda i: (i, 0)),
        out_shape=jax.ShapeDtypeStruct((N, H), jnp.float32),
    )(out_raw, st2, g2.reshape(1, H), beta2.reshape(1, H))

    return (feat, out_feat)


# speedup vs baseline: 9.5609x; 1.3734x over previous
import functools

import jax
import jax.numpy as jnp
from jax import lax
from jax.experimental import pallas as pl
from jax.experimental.pallas import tpu as pltpu
from jax.experimental.pallas import tpu_sc as plsc

N, E, D, H = 10000, 320000, 128, 128
DG = 16
K = 50
NCORES = 2
NSUB = 16
NTILES = NCORES * NSUB
EPT = E // NTILES
CH = EPT // K
RPS = N // NSUB
BM = 2000
NB = N // BM
EPS = 1e-5


def _lin_stats_body(x_ref, w_ref, b_ref, h_ref, st_ref, acc_ref):
    i = pl.program_id(0)

    @pl.when(i == 0)
    def _():
        acc_ref[...] = jnp.zeros_like(acc_ref)

    h = jnp.dot(x_ref[...], w_ref[...], preferred_element_type=jnp.float32)
    h = h + b_ref[...]
    h_ref[...] = h
    acc_ref[0:1, :] += jnp.sum(h, axis=0, keepdims=True)
    acc_ref[1:2, :] += jnp.sum(h * h, axis=0, keepdims=True)

    @pl.when(i == NB - 1)
    def _():
        st_ref[...] = acc_ref[...]


def _bn_relu_body(h_ref, st_ref, g_ref, be_ref, f_ref):
    m = st_ref[0:1, :] / N
    v = st_ref[1:2, :] / N - m * m
    inv = g_ref[...] * lax.rsqrt(v + EPS)
    f_ref[...] = jnp.maximum((h_ref[...] - m) * inv + be_ref[...], 0.0)


def _sc_agg_body(feat_hbm, edge_hbm, ones_hbm, zagg_hbm, zdeg_hbm,
                 agg_hbm, deg_hbm,
                 src_v, dst_v, ones_v, rows0_v, rows1_v, agg_sh, deg_sh,
                 sem0, sem1):
    c = lax.axis_index("c")
    s = lax.axis_index("s")
    wid = c * NSUB + s
    pltpu.sync_copy(edge_hbm.at[0, wid], src_v)
    pltpu.sync_copy(edge_hbm.at[1, wid], dst_v)
    pltpu.sync_copy(ones_hbm, ones_v)
    pltpu.sync_copy(zagg_hbm, agg_sh.at[pl.ds(s * RPS, RPS)])
    pltpu.sync_copy(zdeg_hbm, deg_sh.at[pl.ds(s * RPS, RPS)])
    plsc.subcore_barrier()

    pltpu.async_copy(feat_hbm.at[src_v.at[0]], rows0_v, sem0)

    def body(t, carry):
        j0 = 2 * t
        pltpu.async_copy(feat_hbm.at[src_v.at[j0 + 1]], rows1_v, sem1)
        pltpu.make_async_copy(feat_hbm.at[src_v.at[j0]], rows0_v, sem0).wait()
        pltpu.sync_copy(rows0_v, agg_sh.at[dst_v.at[j0]], add=True)
        pltpu.sync_copy(ones_v, deg_sh.at[dst_v.at[j0]], add=True)

        @pl.when(j0 + 2 < CH)
        def _():
            pltpu.async_copy(feat_hbm.at[src_v.at[j0 + 2]], rows0_v, sem0)

        pltpu.make_async_copy(feat_hbm.at[src_v.at[j0 + 1]], rows1_v, sem1).wait()
        pltpu.sync_copy(rows1_v, agg_sh.at[dst_v.at[j0 + 1]], add=True)
        pltpu.sync_copy(ones_v, deg_sh.at[dst_v.at[j0 + 1]], add=True)
        return carry

    lax.fori_loop(0, CH // 2, body, 0)
    plsc.subcore_barrier()
    pltpu.sync_copy(agg_sh.at[pl.ds(s * RPS, RPS)],
                    agg_hbm.at[c, pl.ds(s * RPS, RPS)])
    pltpu.sync_copy(deg_sh.at[pl.ds(s * RPS, RPS)],
                    deg_hbm.at[c, pl.ds(s * RPS, RPS)])


def _make_sc_agg():
    mesh = plsc.VectorSubcoreMesh(core_axis_name="c", subcore_axis_name="s")
    return pl.kernel(
        _sc_agg_body,
        out_type=(
            jax.ShapeDtypeStruct((NCORES, N, D), jnp.float32),
            jax.ShapeDtypeStruct((NCORES, N, DG), jnp.float32),
        ),
        mesh=mesh,
        scratch_types=[
            pltpu.VMEM((CH, K), jnp.int32),
            pltpu.VMEM((CH, K), jnp.int32),
            pltpu.VMEM((K, DG), jnp.float32),
            pltpu.VMEM((K, D), jnp.float32),
            pltpu.VMEM((K, D), jnp.float32),
            pltpu.VMEM_SHARED((N, D), jnp.float32),
            pltpu.VMEM_SHARED((N, DG), jnp.float32),
            pltpu.SemaphoreType.DMA,
            pltpu.SemaphoreType.DMA,
        ],
        compiler_params=pltpu.CompilerParams(use_tc_tiling_on_sc=False),
    )


def _conv_stats_body(a0_ref, a1_ref, d0_ref, d1_ref, f_ref, wl_ref, wr_ref,
                     bl_ref, o_ref, st_ref, acc_ref):
    i = pl.program_id(0)

    @pl.when(i == 0)
    def _():
        acc_ref[...] = jnp.zeros_like(acc_ref)

    a = a0_ref[0] + a1_ref[0]
    deg = jnp.sum(d0_ref[0] + d1_ref[0], axis=1, keepdims=True) * (1.0 / DG)
    mean = a / jnp.maximum(deg, 1.0)
    o = jnp.dot(mean, wl_ref[...], preferred_element_type=jnp.float32)
    o = o + jnp.dot(f_ref[...], wr_ref[...], preferred_element_type=jnp.float32)
    o = o + bl_ref[...]
    o_ref[...] = o
    acc_ref[0:1, :] += jnp.sum(o, axis=0, keepdims=True)
    acc_ref[1:2, :] += jnp.sum(o * o, axis=0, keepdims=True)

    @pl.when(i == NB - 1)
    def _():
        st_ref[...] = acc_ref[...]


def _bn_body(o_ref, st_ref, g_ref, be_ref, out_ref):
    m = st_ref[0:1, :] / N
    v = st_ref[1:2, :] / N - m * m
    inv = g_ref[...] * lax.rsqrt(v + EPS)
    out_ref[...] = (o_ref[...] - m) * inv + be_ref[...]


def kernel(x, edge_index, W1, b1, g1, beta1, Wl, bl, Wr, g2, beta2):
    er = edge_index.reshape(2, NTILES, CH, K)

    h, st1 = pl.pallas_call(
        _lin_stats_body,
        grid=(NB,),
        in_specs=[
            pl.BlockSpec((BM, D), lambda i: (i, 0)),
            pl.BlockSpec((D, H), lambda i: (0, 0)),
            pl.BlockSpec((1, H), lambda i: (0, 0)),
        ],
        out_specs=[
            pl.BlockSpec((BM, H), lambda i: (i, 0)),
            pl.BlockSpec((8, H), lambda i: (0, 0)),
        ],
        out_shape=[
            jax.ShapeDtypeStruct((N, H), jnp.float32),
            jax.ShapeDtypeStruct((8, H), jnp.float32),
        ],
        scratch_shapes=[pltpu.VMEM((8, H), jnp.float32)],
    )(x, W1, b1.reshape(1, H))

    feat = pl.pallas_call(
        _bn_relu_body,
        grid=(NB,),
        in_specs=[
            pl.BlockSpec((BM, H), lambda i: (i, 0)),
            pl.BlockSpec((8, H), lambda i: (0, 0)),
            pl.BlockSpec((1, H), lambda i: (0, 0)),
            pl.BlockSpec((1, H), lambda i: (0, 0)),
        ],
        out_specs=pl.BlockSpec((BM, H), lambda i: (i, 0)),
        out_shape=jax.ShapeDtypeStruct((N, H), jnp.float32),
    )(h, st1, g1.reshape(1, H), beta1.reshape(1, H))

    ones = jnp.ones((K, DG), jnp.float32)
    zagg = jnp.zeros((RPS, D), jnp.float32)
    zdeg = jnp.zeros((RPS, DG), jnp.float32)
    aggp, degp = _make_sc_agg()(feat, er, ones, zagg, zdeg)

    out_raw, st2 = pl.pallas_call(
        _conv_stats_body,
        grid=(NB,),
        in_specs=[
            pl.BlockSpec((1, BM, D), lambda i: (0, i, 0)),
            pl.BlockSpec((1, BM, D), lambda i: (1, i, 0)),
            pl.BlockSpec((1, BM, DG), lambda i: (0, i, 0)),
            pl.BlockSpec((1, BM, DG), lambda i: (1, i, 0)),
            pl.BlockSpec((BM, H), lambda i: (i, 0)),
            pl.BlockSpec((H, H), lambda i: (0, 0)),
            pl.BlockSpec((H, H), lambda i: (0, 0)),
            pl.BlockSpec((1, H), lambda i: (0, 0)),
        ],
        out_specs=[
            pl.BlockSpec((BM, H), lambda i: (i, 0)),
            pl.BlockSpec((8, H), lambda i: (0, 0)),
        ],
        out_shape=[
            jax.ShapeDtypeStruct((N, H), jnp.float32),
            jax.ShapeDtypeStruct((8, H), jnp.float32),
        ],
        scratch_shapes=[pltpu.VMEM((8, H), jnp.float32)],
    )(aggp, aggp, degp, degp, feat, Wl, Wr, bl.reshape(1, H))

    out_feat = pl.pallas_call(
        _bn_body,
        grid=(NB,),
        in_specs=[
            pl.BlockSpec((BM, H), lambda i: (i, 0)),
            pl.BlockSpec((8, H), lambda i: (0, 0)),
            pl.BlockSpec((1, H), lambda i: (0, 0)),
            pl.BlockSpec((1, H), lambda i: (0, 0)),
        ],
        out_specs=pl.BlockSpec((BM, H), lambda i: (i, 0)),
        out_shape=jax.ShapeDtypeStruct((N, H), jnp.float32),
    )(out_raw, st2, g2.reshape(1, H), beta2.reshape(1, H))

    return (feat, out_feat)


# speedup vs baseline: 11.8967x; 1.2443x over previous
import functools

import jax
import jax.numpy as jnp
from jax import lax
from jax.experimental import pallas as pl
from jax.experimental.pallas import tpu as pltpu
from jax.experimental.pallas import tpu_sc as plsc

N, E, D, H = 10000, 320000, 128, 128
DG = 16
K = 128
NCORES = 2
NSUB = 16
NTILES = NCORES * NSUB
EPT = 10240
EPAD = NTILES * EPT - E
CH = EPT // K
NPH = 4
CPP = CH // NPH
NP = 10016
RPS = NP // NSUB
BM = 2000
NB = N // BM
EPS = 1e-5


def _lin_stats_body(x_ref, w_ref, b_ref, h_ref, st_ref, acc_ref):
    i = pl.program_id(0)

    @pl.when(i == 0)
    def _():
        acc_ref[...] = jnp.zeros_like(acc_ref)

    h = jnp.dot(x_ref[...], w_ref[...], preferred_element_type=jnp.float32)
    h = h + b_ref[...]
    h_ref[...] = h
    acc_ref[0:1, :] += jnp.sum(h, axis=0, keepdims=True)
    acc_ref[1:2, :] += jnp.sum(h * h, axis=0, keepdims=True)

    @pl.when(i == NB - 1)
    def _():
        st_ref[...] = acc_ref[...]


def _bn_relu_body(h_ref, st_ref, g_ref, be_ref, f_ref):
    m = st_ref[0:1, :] / N
    v = st_ref[1:2, :] / N - m * m
    inv = g_ref[...] * lax.rsqrt(v + EPS)
    f_ref[...] = jnp.maximum((h_ref[...] - m) * inv + be_ref[...], 0.0)


def _fr_body(f_ref, wr_ref, bl_ref, o_ref):
    o = jnp.dot(f_ref[...], wr_ref[...], preferred_element_type=jnp.float32)
    o_ref[...] = o + bl_ref[...]


def _sc_agg_body(feat_hbm, edge_hbm, ones_hbm, zagg_hbm, zdeg_hbm,
                 agg_hbm, deg_hbm,
                 src_v, dst_v, ones_v, rows0_v, rows1_v, agg_sh, deg_sh,
                 sem0, sem1):
    c = lax.axis_index("c")
    s = lax.axis_index("s")
    wid = c * NSUB + s
    pltpu.sync_copy(ones_hbm, ones_v)
    pltpu.sync_copy(zagg_hbm, agg_sh.at[pl.ds(s * RPS, RPS)])
    pltpu.sync_copy(zdeg_hbm, deg_sh.at[pl.ds(s * RPS, RPS)])
    plsc.subcore_barrier()

    for p in range(NPH):
        pltpu.sync_copy(edge_hbm.at[0, wid, pl.ds(p * CPP, CPP)], src_v)
        pltpu.sync_copy(edge_hbm.at[1, wid, pl.ds(p * CPP, CPP)], dst_v)
        pltpu.async_copy(feat_hbm.at[src_v.at[0]], rows0_v, sem0)

        def body(t, carry):
            j0 = 2 * t
            pltpu.async_copy(feat_hbm.at[src_v.at[j0 + 1]], rows1_v, sem1)
            pltpu.make_async_copy(feat_hbm.at[src_v.at[j0]], rows0_v,
                                  sem0).wait()
            pltpu.sync_copy(rows0_v, agg_sh.at[dst_v.at[j0]], add=True)
            pltpu.sync_copy(ones_v, deg_sh.at[dst_v.at[j0]], add=True)

            @pl.when(j0 + 2 < CPP)
            def _():
                pltpu.async_copy(feat_hbm.at[src_v.at[j0 + 2]], rows0_v, sem0)

            pltpu.make_async_copy(feat_hbm.at[src_v.at[j0 + 1]], rows1_v,
                                  sem1).wait()
            pltpu.sync_copy(rows1_v, agg_sh.at[dst_v.at[j0 + 1]], add=True)
            pltpu.sync_copy(ones_v, deg_sh.at[dst_v.at[j0 + 1]], add=True)
            return carry

        lax.fori_loop(0, CPP // 2, body, 0)

    plsc.subcore_barrier()
    pltpu.sync_copy(agg_sh.at[pl.ds(s * RPS, RPS)],
                    agg_hbm.at[c, pl.ds(s * RPS, RPS)])
    pltpu.sync_copy(deg_sh.at[pl.ds(s * RPS, RPS)],
                    deg_hbm.at[c, pl.ds(s * RPS, RPS)])


def _make_sc_agg():
    mesh = plsc.VectorSubcoreMesh(core_axis_name="c", subcore_axis_name="s")
    return pl.kernel(
        _sc_agg_body,
        out_type=(
            jax.ShapeDtypeStruct((NCORES, NP, D), jnp.float32),
            jax.ShapeDtypeStruct((NCORES, NP, DG), jnp.float32),
        ),
        mesh=mesh,
        scratch_types=[
            pltpu.VMEM((CPP, K), jnp.int32),
            pltpu.VMEM((CPP, K), jnp.int32),
            pltpu.VMEM((K, DG), jnp.float32),
            pltpu.VMEM((K, D), jnp.float32),
            pltpu.VMEM((K, D), jnp.float32),
            pltpu.VMEM_SHARED((NP, D), jnp.float32),
            pltpu.VMEM_SHARED((NP, DG), jnp.float32),
            pltpu.SemaphoreType.DMA,
            pltpu.SemaphoreType.DMA,
        ],
        compiler_params=pltpu.CompilerParams(use_tc_tiling_on_sc=False),
    )


def _conv_stats_body(a0_ref, a1_ref, d0_ref, d1_ref, fr_ref, wl_ref,
                     o_ref, st_ref, acc_ref):
    i = pl.program_id(0)

    @pl.when(i == 0)
    def _():
        acc_ref[...] = jnp.zeros_like(acc_ref)

    a = a0_ref[0] + a1_ref[0]
    deg = jnp.sum(d0_ref[0] + d1_ref[0], axis=1, keepdims=True) * (1.0 / DG)
    mean = a / jnp.maximum(deg, 1.0)
    o = jnp.dot(mean, wl_ref[...], preferred_element_type=jnp.float32)
    o = o + fr_ref[...]
    o_ref[...] = o
    acc_ref[0:1, :] += jnp.sum(o, axis=0, keepdims=True)
    acc_ref[1:2, :] += jnp.sum(o * o, axis=0, keepdims=True)

    @pl.when(i == NB - 1)
    def _():
        st_ref[...] = acc_ref[...]


def _bn_body(o_ref, st_ref, g_ref, be_ref, out_ref):
    m = st_ref[0:1, :] / N
    v = st_ref[1:2, :] / N - m * m
    inv = g_ref[...] * lax.rsqrt(v + EPS)
    out_ref[...] = (o_ref[...] - m) * inv + be_ref[...]


def kernel(x, edge_index, W1, b1, g1, beta1, Wl, bl, Wr, g2, beta2):
    ar = jnp.arange(EPAD, dtype=jnp.int32)
    pad = jnp.stack([ar * 41 % N, N + (ar % (NP - N))])
    er = jnp.concatenate([edge_index, pad], axis=1)
    er = er.reshape(2, NTILES, CH, K)

    h, st1 = pl.pallas_call(
        _lin_stats_body,
        grid=(NB,),
        in_specs=[
            pl.BlockSpec((BM, D), lambda i: (i, 0)),
            pl.BlockSpec((D, H), lambda i: (0, 0)),
            pl.BlockSpec((1, H), lambda i: (0, 0)),
        ],
        out_specs=[
            pl.BlockSpec((BM, H), lambda i: (i, 0)),
            pl.BlockSpec((8, H), lambda i: (0, 0)),
        ],
        out_shape=[
            jax.ShapeDtypeStruct((N, H), jnp.float32),
            jax.ShapeDtypeStruct((8, H), jnp.float32),
        ],
        scratch_shapes=[pltpu.VMEM((8, H), jnp.float32)],
    )(x, W1, b1.reshape(1, H))

    feat = pl.pallas_call(
        _bn_relu_body,
        grid=(NB,),
        in_specs=[
            pl.BlockSpec((BM, H), lambda i: (i, 0)),
            pl.BlockSpec((8, H), lambda i: (0, 0)),
            pl.BlockSpec((1, H), lambda i: (0, 0)),
            pl.BlockSpec((1, H), lambda i: (0, 0)),
        ],
        out_specs=pl.BlockSpec((BM, H), lambda i: (i, 0)),
        out_shape=jax.ShapeDtypeStruct((N, H), jnp.float32),
    )(h, st1, g1.reshape(1, H), beta1.reshape(1, H))

    ones = jnp.ones((K, DG), jnp.float32)
    zagg = jnp.zeros((RPS, D), jnp.float32)
    zdeg = jnp.zeros((RPS, DG), jnp.float32)
    aggp, degp = _make_sc_agg()(feat, er, ones, zagg, zdeg)

    fr = pl.pallas_call(
        _fr_body,
        grid=(NB,),
        in_specs=[
            pl.BlockSpec((BM, H), lambda i: (i, 0)),
            pl.BlockSpec((H, H), lambda i: (0, 0)),
            pl.BlockSpec((1, H), lambda i: (0, 0)),
        ],
        out_specs=pl.BlockSpec((BM, H), lambda i: (i, 0)),
        out_shape=jax.ShapeDtypeStruct((N, H), jnp.float32),
    )(feat, Wr, bl.reshape(1, H))

    out_raw, st2 = pl.pallas_call(
        _conv_stats_body,
        grid=(NB,),
        in_specs=[
            pl.BlockSpec((1, BM, D), lambda i: (0, i, 0)),
            pl.BlockSpec((1, BM, D), lambda i: (1, i, 0)),
            pl.BlockSpec((1, BM, DG), lambda i: (0, i, 0)),
            pl.BlockSpec((1, BM, DG), lambda i: (1, i, 0)),
            pl.BlockSpec((BM, H), lambda i: (i, 0)),
            pl.BlockSpec((H, H), lambda i: (0, 0)),
        ],
        out_specs=[
            pl.BlockSpec((BM, H), lambda i: (i, 0)),
            pl.BlockSpec((8, H), lambda i: (0, 0)),
        ],
        out_shape=[
            jax.ShapeDtypeStruct((N, H), jnp.float32),
            jax.ShapeDtypeStruct((8, H), jnp.float32),
        ],
        scratch_shapes=[pltpu.VMEM((8, H), jnp.float32)],
    )(aggp, aggp, degp, degp, fr, Wl)

    out_feat = pl.pallas_call(
        _bn_body,
        grid=(NB,),
        in_specs=[
            pl.BlockSpec((BM, H), lambda i: (i, 0)),
            pl.BlockSpec((8, H), lambda i: (0, 0)),
            pl.BlockSpec((1, H), lambda i: (0, 0)),
            pl.BlockSpec((1, H), lambda i: (0, 0)),
        ],
        out_specs=pl.BlockSpec((BM, H), lambda i: (i, 0)),
        out_shape=jax.ShapeDtypeStruct((N, H), jnp.float32),
    )(out_raw, st2, g2.reshape(1, H), beta2.reshape(1, H))

    return (feat, out_feat)


# speedup vs baseline: 12.0288x; 1.0111x over previous
import functools

import jax
import jax.numpy as jnp
from jax import lax
from jax.experimental import pallas as pl
from jax.experimental.pallas import tpu as pltpu
from jax.experimental.pallas import tpu_sc as plsc

N, E, D, H = 10000, 320000, 128, 128
DG = 16
K = 128
NCORES = 2
NSUB = 16
NTILES = NCORES * NSUB
EPT = 10240
EPAD = NTILES * EPT - E
CH = EPT // K
NPH = 4
CPP = CH // NPH
NP = 10016
RPS = NP // NSUB
BM = 2000
NB = N // BM
EPS = 1e-5


def _lin_stats_body(x_ref, w_ref, b_ref, h_ref, st_ref, acc_ref):
    i = pl.program_id(0)

    @pl.when(i == 0)
    def _():
        acc_ref[...] = jnp.zeros_like(acc_ref)

    h = jnp.dot(x_ref[...], w_ref[...], preferred_element_type=jnp.float32)
    h = h + b_ref[...]
    h_ref[...] = h
    acc_ref[0:1, :] += jnp.sum(h, axis=0, keepdims=True)
    acc_ref[1:2, :] += jnp.sum(h * h, axis=0, keepdims=True)

    @pl.when(i == NB - 1)
    def _():
        st_ref[...] = acc_ref[...]


def _bn_relu_body(h_ref, st_ref, g_ref, be_ref, f_ref):
    m = st_ref[0:1, :] / N
    v = st_ref[1:2, :] / N - m * m
    inv = g_ref[...] * lax.rsqrt(v + EPS)
    f_ref[...] = jnp.maximum((h_ref[...] - m) * inv + be_ref[...], 0.0)


def _fr_body(f_ref, wr_ref, bl_ref, o_ref):
    o = jnp.dot(f_ref[...], wr_ref[...], preferred_element_type=jnp.float32)
    o_ref[...] = o + bl_ref[...]


def _sc_agg_body(feat_hbm, edge_hbm, ones_hbm, zagg_hbm, zdeg_hbm,
                 agg_hbm, deg_hbm,
                 src_v, dst_v, ones_v, rows0_v, rows1_v, agg_sh, deg_sh,
                 sem0, sem1, semd):
    c = lax.axis_index("c")
    s = lax.axis_index("s")
    wid = c * NSUB + s
    pltpu.sync_copy(ones_hbm, ones_v)
    pltpu.sync_copy(zagg_hbm, agg_sh.at[pl.ds(s * RPS, RPS)])
    pltpu.sync_copy(zdeg_hbm, deg_sh.at[pl.ds(s * RPS, RPS)])
    plsc.subcore_barrier()

    for p in range(NPH):
        pltpu.sync_copy(edge_hbm.at[0, wid, pl.ds(p * CPP, CPP)], src_v)
        pltpu.sync_copy(edge_hbm.at[1, wid, pl.ds(p * CPP, CPP)], dst_v)
        pltpu.async_copy(feat_hbm.at[src_v.at[0]], rows0_v, sem0)

        def body(t, carry):
            j0 = 2 * t
            pltpu.async_copy(feat_hbm.at[src_v.at[j0 + 1]], rows1_v, sem1)
            pltpu.make_async_copy(feat_hbm.at[src_v.at[j0]], rows0_v,
                                  sem0).wait()
            pltpu.sync_copy(rows0_v, agg_sh.at[dst_v.at[j0]], add=True)
            pltpu.async_copy(ones_v, deg_sh.at[dst_v.at[j0]], semd, add=True)

            @pl.when(j0 + 2 < CPP)
            def _():
                pltpu.async_copy(feat_hbm.at[src_v.at[j0 + 2]], rows0_v, sem0)

            pltpu.make_async_copy(feat_hbm.at[src_v.at[j0 + 1]], rows1_v,
                                  sem1).wait()
            pltpu.sync_copy(rows1_v, agg_sh.at[dst_v.at[j0 + 1]], add=True)
            pltpu.async_copy(ones_v, deg_sh.at[dst_v.at[j0 + 1]], semd, add=True)
            return carry

        lax.fori_loop(0, CPP // 2, body, 0)

    def drain(j, carry):
        pltpu.make_async_copy(ones_hbm, ones_v, semd).wait()
        return carry

    lax.fori_loop(0, CH, drain, 0)
    plsc.subcore_barrier()
    pltpu.sync_copy(agg_sh.at[pl.ds(s * RPS, RPS)],
                    agg_hbm.at[c, pl.ds(s * RPS, RPS)])
    pltpu.sync_copy(deg_sh.at[pl.ds(s * RPS, RPS)],
                    deg_hbm.at[c, pl.ds(s * RPS, RPS)])


def _make_sc_agg():
    mesh = plsc.VectorSubcoreMesh(core_axis_name="c", subcore_axis_name="s")
    return pl.kernel(
        _sc_agg_body,
        out_type=(
            jax.ShapeDtypeStruct((NCORES, NP, D), jnp.float32),
            jax.ShapeDtypeStruct((NCORES, NP, DG), jnp.float32),
        ),
        mesh=mesh,
        scratch_types=[
            pltpu.VMEM((CPP, K), jnp.int32),
            pltpu.VMEM((CPP, K), jnp.int32),
            pltpu.VMEM((K, DG), jnp.float32),
            pltpu.VMEM((K, D), jnp.float32),
            pltpu.VMEM((K, D), jnp.float32),
            pltpu.VMEM_SHARED((NP, D), jnp.float32),
            pltpu.VMEM_SHARED((NP, DG), jnp.float32),
            pltpu.SemaphoreType.DMA,
            pltpu.SemaphoreType.DMA,
            pltpu.SemaphoreType.DMA,
        ],
        compiler_params=pltpu.CompilerParams(use_tc_tiling_on_sc=False),
    )


def _conv_stats_body(a0_ref, a1_ref, d0_ref, d1_ref, fr_ref, wl_ref,
                     o_ref, st_ref, acc_ref):
    i = pl.program_id(0)

    @pl.when(i == 0)
    def _():
        acc_ref[...] = jnp.zeros_like(acc_ref)

    a = a0_ref[0] + a1_ref[0]
    deg = jnp.sum(d0_ref[0] + d1_ref[0], axis=1, keepdims=True) * (1.0 / DG)
    mean = a / jnp.maximum(deg, 1.0)
    o = jnp.dot(mean, wl_ref[...], preferred_element_type=jnp.float32)
    o = o + fr_ref[...]
    o_ref[...] = o
    acc_ref[0:1, :] += jnp.sum(o, axis=0, keepdims=True)
    acc_ref[1:2, :] += jnp.sum(o * o, axis=0, keepdims=True)

    @pl.when(i == NB - 1)
    def _():
        st_ref[...] = acc_ref[...]


def _bn_body(o_ref, st_ref, g_ref, be_ref, out_ref):
    m = st_ref[0:1, :] / N
    v = st_ref[1:2, :] / N - m * m
    inv = g_ref[...] * lax.rsqrt(v + EPS)
    out_ref[...] = (o_ref[...] - m) * inv + be_ref[...]


def kernel(x, edge_index, W1, b1, g1, beta1, Wl, bl, Wr, g2, beta2):
    ar = jnp.arange(EPAD, dtype=jnp.int32)
    pad = jnp.stack([ar * 41 % N, N + (ar % (NP - N))])
    er = jnp.concatenate([edge_index, pad], axis=1)
    er = er.reshape(2, NTILES, CH, K)

    h, st1 = pl.pallas_call(
        _lin_stats_body,
        grid=(NB,),
        in_specs=[
            pl.BlockSpec((BM, D), lambda i: (i, 0)),
            pl.BlockSpec((D, H), lambda i: (0, 0)),
            pl.BlockSpec((1, H), lambda i: (0, 0)),
        ],
        out_specs=[
            pl.BlockSpec((BM, H), lambda i: (i, 0)),
            pl.BlockSpec((8, H), lambda i: (0, 0)),
        ],
        out_shape=[
            jax.ShapeDtypeStruct((N, H), jnp.float32),
            jax.ShapeDtypeStruct((8, H), jnp.float32),
        ],
        scratch_shapes=[pltpu.VMEM((8, H), jnp.float32)],
    )(x, W1, b1.reshape(1, H))

    feat = pl.pallas_call(
        _bn_relu_body,
        grid=(NB,),
        in_specs=[
            pl.BlockSpec((BM, H), lambda i: (i, 0)),
            pl.BlockSpec((8, H), lambda i: (0, 0)),
            pl.BlockSpec((1, H), lambda i: (0, 0)),
            pl.BlockSpec((1, H), lambda i: (0, 0)),
        ],
        out_specs=pl.BlockSpec((BM, H), lambda i: (i, 0)),
        out_shape=jax.ShapeDtypeStruct((N, H), jnp.float32),
    )(h, st1, g1.reshape(1, H), beta1.reshape(1, H))

    ones = jnp.ones((K, DG), jnp.float32)
    zagg = jnp.zeros((RPS, D), jnp.float32)
    zdeg = jnp.zeros((RPS, DG), jnp.float32)
    aggp, degp = _make_sc_agg()(feat, er, ones, zagg, zdeg)

    fr = pl.pallas_call(
        _fr_body,
        grid=(NB,),
        in_specs=[
            pl.BlockSpec((BM, H), lambda i: (i, 0)),
            pl.BlockSpec((H, H), lambda i: (0, 0)),
            pl.BlockSpec((1, H), lambda i: (0, 0)),
        ],
        out_specs=pl.BlockSpec((BM, H), lambda i: (i, 0)),
        out_shape=jax.ShapeDtypeStruct((N, H), jnp.float32),
    )(feat, Wr, bl.reshape(1, H))

    out_raw, st2 = pl.pallas_call(
        _conv_stats_body,
        grid=(NB,),
        in_specs=[
            pl.BlockSpec((1, BM, D), lambda i: (0, i, 0)),
            pl.BlockSpec((1, BM, D), lambda i: (1, i, 0)),
            pl.BlockSpec((1, BM, DG), lambda i: (0, i, 0)),
            pl.BlockSpec((1, BM, DG), lambda i: (1, i, 0)),
            pl.BlockSpec((BM, H), lambda i: (i, 0)),
            pl.BlockSpec((H, H), lambda i: (0, 0)),
        ],
        out_specs=[
            pl.BlockSpec((BM, H), lambda i: (i, 0)),
            pl.BlockSpec((8, H), lambda i: (0, 0)),
        ],
        out_shape=[
            jax.ShapeDtypeStruct((N, H), jnp.float32),
            jax.ShapeDtypeStruct((8, H), jnp.float32),
        ],
        scratch_shapes=[pltpu.VMEM((8, H), jnp.float32)],
    )(aggp, aggp, degp, degp, fr, Wl)

    out_feat = pl.pallas_call(
        _bn_body,
        grid=(NB,),
        in_specs=[
            pl.BlockSpec((BM, H), lambda i: (i, 0)),
            pl.BlockSpec((8, H), lambda i: (0, 0)),
            pl.BlockSpec((1, H), lambda i: (0, 0)),
            pl.BlockSpec((1, H), lambda i: (0, 0)),
        ],
        out_specs=pl.BlockSpec((BM, H), lambda i: (i, 0)),
        out_shape=jax.ShapeDtypeStruct((N, H), jnp.float32),
    )(out_raw, st2, g2.reshape(1, H), beta2.reshape(1, H))

    return (feat, out_feat)


# speedup vs baseline: 12.6174x; 1.0489x over previous
import functools

import jax
import jax.numpy as jnp
from jax import lax
from jax.experimental import pallas as pl
from jax.experimental.pallas import tpu as pltpu
from jax.experimental.pallas import tpu_sc as plsc

N, E, D, H = 10000, 320000, 128, 128
DG = 16
K = 128
NCORES = 2
NSUB = 16
NTILES = NCORES * NSUB
EPT = 10240
EPAD = NTILES * EPT - E
CH = EPT // K
NPH = 4
CPP = CH // NPH
NP = 10016
RPS = NP // NSUB
BM = 2000
NB = N // BM
EPS = 1e-5


def _lin_bn_relu_body(x_ref, w_ref, b_ref, g_ref, be_ref, f_ref,
                      h_ref, acc_ref):
    i = pl.program_id(0)

    @pl.when(i == 0)
    def _():
        acc_ref[...] = jnp.zeros_like(acc_ref)

    @pl.when(i < NB)
    def _():
        h = jnp.dot(x_ref[...], w_ref[...],
                    preferred_element_type=jnp.float32)
        h = h + b_ref[...]
        h_ref[pl.ds((i % NB) * BM, BM), :] = h
        acc_ref[0:1, :] += jnp.sum(h, axis=0, keepdims=True)
        acc_ref[1:2, :] += jnp.sum(h * h, axis=0, keepdims=True)

    @pl.when(i >= NB)
    def _():
        m = acc_ref[0:1, :] / N
        v = acc_ref[1:2, :] / N - m * m
        inv = g_ref[...] * lax.rsqrt(v + EPS)
        h = h_ref[pl.ds((i % NB) * BM, BM), :]
        f_ref[...] = jnp.maximum((h - m) * inv + be_ref[...], 0.0)


def _fr_body(f_ref, wr_ref, bl_ref, o_ref):
    o = jnp.dot(f_ref[...], wr_ref[...], preferred_element_type=jnp.float32)
    o_ref[...] = o + bl_ref[...]


def _sc_agg_body(feat_hbm, edge_hbm, ones_hbm, zagg_hbm, zdeg_hbm,
                 agg_hbm, deg_hbm,
                 src_v, dst_v, ones_v, rows0_v, rows1_v, agg_sh, deg_sh,
                 sem0, sem1, semd, semz):
    c = lax.axis_index("c")
    s = lax.axis_index("s")
    wid = c * NSUB + s
    zagg_cp = pltpu.async_copy(zagg_hbm, agg_sh.at[pl.ds(s * RPS, RPS)], semz)
    zdeg_cp = pltpu.async_copy(zdeg_hbm, deg_sh.at[pl.ds(s * RPS, RPS)], semz)
    pltpu.sync_copy(ones_hbm, ones_v)

    for p in range(NPH):
        pltpu.sync_copy(edge_hbm.at[0, wid, pl.ds(p * CPP, CPP)], src_v)
        pltpu.sync_copy(edge_hbm.at[1, wid, pl.ds(p * CPP, CPP)], dst_v)
        pltpu.async_copy(feat_hbm.at[src_v.at[0]], rows0_v, sem0)
        if p == 0:
            zagg_cp.wait()
            zdeg_cp.wait()
            plsc.subcore_barrier()

        def body(t, carry):
            j0 = 2 * t
            pltpu.async_copy(feat_hbm.at[src_v.at[j0 + 1]], rows1_v, sem1)
            pltpu.make_async_copy(feat_hbm.at[src_v.at[j0]], rows0_v,
                                  sem0).wait()
            pltpu.sync_copy(rows0_v, agg_sh.at[dst_v.at[j0]], add=True)
            pltpu.async_copy(ones_v, deg_sh.at[dst_v.at[j0]], semd, add=True)

            @pl.when(j0 + 2 < CPP)
            def _():
                pltpu.async_copy(feat_hbm.at[src_v.at[j0 + 2]], rows0_v, sem0)

            pltpu.make_async_copy(feat_hbm.at[src_v.at[j0 + 1]], rows1_v,
                                  sem1).wait()
            pltpu.sync_copy(rows1_v, agg_sh.at[dst_v.at[j0 + 1]], add=True)
            pltpu.async_copy(ones_v, deg_sh.at[dst_v.at[j0 + 1]], semd,
                             add=True)
            return carry

        lax.fori_loop(0, CPP // 2, body, 0)

    def drain(j, carry):
        pltpu.make_async_copy(ones_hbm, ones_v, semd).wait()
        return carry

    lax.fori_loop(0, CH, drain, 0)
    plsc.subcore_barrier()
    pltpu.sync_copy(agg_sh.at[pl.ds(s * RPS, RPS)],
                    agg_hbm.at[c, pl.ds(s * RPS, RPS)])
    pltpu.sync_copy(deg_sh.at[pl.ds(s * RPS, RPS)],
                    deg_hbm.at[c, pl.ds(s * RPS, RPS)])


def _make_sc_agg():
    mesh = plsc.VectorSubcoreMesh(core_axis_name="c", subcore_axis_name="s")
    return pl.kernel(
        _sc_agg_body,
        out_type=(
            jax.ShapeDtypeStruct((NCORES, NP, D), jnp.float32),
            jax.ShapeDtypeStruct((NCORES, NP, DG), jnp.float32),
        ),
        mesh=mesh,
        scratch_types=[
            pltpu.VMEM((CPP, K), jnp.int32),
            pltpu.VMEM((CPP, K), jnp.int32),
            pltpu.VMEM((K, DG), jnp.float32),
            pltpu.VMEM((K, D), jnp.float32),
            pltpu.VMEM((K, D), jnp.float32),
            pltpu.VMEM_SHARED((NP, D), jnp.float32),
            pltpu.VMEM_SHARED((NP, DG), jnp.float32),
            pltpu.SemaphoreType.DMA,
            pltpu.SemaphoreType.DMA,
            pltpu.SemaphoreType.DMA,
            pltpu.SemaphoreType.DMA,
        ],
        compiler_params=pltpu.CompilerParams(use_tc_tiling_on_sc=False),
    )


def _conv_stats_body(a0_ref, a1_ref, d0_ref, d1_ref, fr_ref, wl_ref,
                     o_ref, st_ref, acc_ref):
    i = pl.program_id(0)

    @pl.when(i == 0)
    def _():
        acc_ref[...] = jnp.zeros_like(acc_ref)

    a = a0_ref[0] + a1_ref[0]
    deg = jnp.sum(d0_ref[0] + d1_ref[0], axis=1, keepdims=True) * (1.0 / DG)
    mean = a / jnp.maximum(deg, 1.0)
    o = jnp.dot(mean, wl_ref[...], preferred_element_type=jnp.float32)
    o = o + fr_ref[...]
    o_ref[...] = o
    acc_ref[0:1, :] += jnp.sum(o, axis=0, keepdims=True)
    acc_ref[1:2, :] += jnp.sum(o * o, axis=0, keepdims=True)

    @pl.when(i == NB - 1)
    def _():
        st_ref[...] = acc_ref[...]


def _bn_body(o_ref, st_ref, g_ref, be_ref, out_ref):
    m = st_ref[0:1, :] / N
    v = st_ref[1:2, :] / N - m * m
    inv = g_ref[...] * lax.rsqrt(v + EPS)
    out_ref[...] = (o_ref[...] - m) * inv + be_ref[...]


def kernel(x, edge_index, W1, b1, g1, beta1, Wl, bl, Wr, g2, beta2):
    ar = jnp.arange(EPAD, dtype=jnp.int32)
    pad = jnp.stack([ar * 41 % N, N + (ar % (NP - N))])
    er = jnp.concatenate([edge_index, pad], axis=1)
    er = er.reshape(2, NTILES, CH, K)

    feat = pl.pallas_call(
        _lin_bn_relu_body,
        grid=(2 * NB,),
        in_specs=[
            pl.BlockSpec((BM, D), lambda i: (jnp.where(i < NB, i, 0), 0)),
            pl.BlockSpec((D, H), lambda i: (0, 0)),
            pl.BlockSpec((1, H), lambda i: (0, 0)),
            pl.BlockSpec((1, H), lambda i: (0, 0)),
            pl.BlockSpec((1, H), lambda i: (0, 0)),
        ],
        out_specs=pl.BlockSpec((BM, H),
                               lambda i: (jnp.where(i < NB, 0, i - NB), 0)),
        out_shape=jax.ShapeDtypeStruct((N, H), jnp.float32),
        scratch_shapes=[
            pltpu.VMEM((N, H), jnp.float32),
            pltpu.VMEM((8, H), jnp.float32),
        ],
    )(x, W1, b1.reshape(1, H), g1.reshape(1, H), beta1.reshape(1, H))

    ones = jnp.ones((K, DG), jnp.float32)
    zagg = jnp.zeros((RPS, D), jnp.float32)
    zdeg = jnp.zeros((RPS, DG), jnp.float32)
    aggp, degp = _make_sc_agg()(feat, er, ones, zagg, zdeg)

    fr = pl.pallas_call(
        _fr_body,
        grid=(NB,),
        in_specs=[
            pl.BlockSpec((BM, H), lambda i: (i, 0)),
            pl.BlockSpec((H, H), lambda i: (0, 0)),
            pl.BlockSpec((1, H), lambda i: (0, 0)),
        ],
        out_specs=pl.BlockSpec((BM, H), lambda i: (i, 0)),
        out_shape=jax.ShapeDtypeStruct((N, H), jnp.float32),
    )(feat, Wr, bl.reshape(1, H))

    out_raw, st2 = pl.pallas_call(
        _conv_stats_body,
        grid=(NB,),
        in_specs=[
            pl.BlockSpec((1, BM, D), lambda i: (0, i, 0)),
            pl.BlockSpec((1, BM, D), lambda i: (1, i, 0)),
            pl.BlockSpec((1, BM, DG), lambda i: (0, i, 0)),
            pl.BlockSpec((1, BM, DG), lambda i: (1, i, 0)),
            pl.BlockSpec((BM, H), lambda i: (i, 0)),
            pl.BlockSpec((H, H), lambda i: (0, 0)),
        ],
        out_specs=[
            pl.BlockSpec((BM, H), lambda i: (i, 0)),
            pl.BlockSpec((8, H), lambda i: (0, 0)),
        ],
        out_shape=[
            jax.ShapeDtypeStruct((N, H), jnp.float32),
            jax.ShapeDtypeStruct((8, H), jnp.float32),
        ],
        scratch_shapes=[pltpu.VMEM((8, H), jnp.float32)],
    )(aggp, aggp, degp, degp, fr, Wl)

    out_feat = pl.pallas_call(
        _bn_body,
        grid=(NB,),
        in_specs=[
            pl.BlockSpec((BM, H), lambda i: (i, 0)),
            pl.BlockSpec((8, H), lambda i: (0, 0)),
            pl.BlockSpec((1, H), lambda i: (0, 0)),
            pl.BlockSpec((1, H), lambda i: (0, 0)),
        ],
        out_specs=pl.BlockSpec((BM, H), lambda i: (i, 0)),
        out_shape=jax.ShapeDtypeStruct((N, H), jnp.float32),
    )(out_raw, st2, g2.reshape(1, H), beta2.reshape(1, H))

    return (feat, out_feat)


# speedup vs baseline: 13.3436x; 1.0576x over previous
import functools

import jax
import jax.numpy as jnp
from jax import lax
from jax.experimental import pallas as pl
from jax.experimental.pallas import tpu as pltpu
from jax.experimental.pallas import tpu_sc as plsc

N, E, D, H = 10000, 320000, 128, 128
DG = 16
K = 128
NCORES = 2
NSUB = 16
NTILES = NCORES * NSUB
EPT = 10240
EPAD = NTILES * EPT - E
CH = EPT // K
NPH = 4
CPP = CH // NPH
NP = 10016
RPS = NP // NSUB
BM = 2000
NB = N // BM
EPS = 1e-5


def _lin_bn_relu_body(x_ref, w_ref, b_ref, g_ref, be_ref, f_ref, fb_ref,
                      h_ref, acc_ref):
    i = pl.program_id(0)

    @pl.when(i == 0)
    def _():
        acc_ref[...] = jnp.zeros_like(acc_ref)

    @pl.when(i < NB)
    def _():
        h = jnp.dot(x_ref[...], w_ref[...],
                    preferred_element_type=jnp.float32)
        h = h + b_ref[...]
        h_ref[pl.ds((i % NB) * BM, BM), :] = h
        acc_ref[0:1, :] += jnp.sum(h, axis=0, keepdims=True)
        acc_ref[1:2, :] += jnp.sum(h * h, axis=0, keepdims=True)

    @pl.when(i >= NB)
    def _():
        m = acc_ref[0:1, :] / N
        v = acc_ref[1:2, :] / N - m * m
        inv = g_ref[...] * lax.rsqrt(v + EPS)
        h = h_ref[pl.ds((i % NB) * BM, BM), :]
        f = jnp.maximum((h - m) * inv + be_ref[...], 0.0)
        f_ref[...] = f
        fb_ref[...] = f.astype(jnp.bfloat16)


def _fr_body(f_ref, wr_ref, bl_ref, o_ref):
    o = jnp.dot(f_ref[...], wr_ref[...], preferred_element_type=jnp.float32)
    o_ref[...] = o + bl_ref[...]


def _sc_agg_body(feat_hbm, edge_hbm, ones_hbm, zagg_hbm, zdeg_hbm,
                 agg_hbm, deg_hbm,
                 src_v, dst_v, ones_v, rows0_v, rows1_v, agg_sh, deg_sh,
                 sem0, sem1, semd, semz):
    c = lax.axis_index("c")
    s = lax.axis_index("s")
    wid = c * NSUB + s
    zagg_cp = pltpu.async_copy(zagg_hbm, agg_sh.at[pl.ds(s * RPS, RPS)], semz)
    zdeg_cp = pltpu.async_copy(zdeg_hbm, deg_sh.at[pl.ds(s * RPS, RPS)], semz)
    pltpu.sync_copy(ones_hbm, ones_v)

    for p in range(NPH):
        pltpu.sync_copy(edge_hbm.at[0, wid, pl.ds(p * CPP, CPP)], src_v)
        pltpu.sync_copy(edge_hbm.at[1, wid, pl.ds(p * CPP, CPP)], dst_v)
        pltpu.async_copy(feat_hbm.at[src_v.at[0]], rows0_v, sem0)
        if p == 0:
            zagg_cp.wait()
            zdeg_cp.wait()
            plsc.subcore_barrier()

        def body(t, carry):
            j0 = 2 * t
            pltpu.async_copy(feat_hbm.at[src_v.at[j0 + 1]], rows1_v, sem1)
            pltpu.make_async_copy(feat_hbm.at[src_v.at[j0]], rows0_v,
                                  sem0).wait()
            pltpu.sync_copy(rows0_v, agg_sh.at[dst_v.at[j0]], add=True)
            pltpu.async_copy(ones_v, deg_sh.at[dst_v.at[j0]], semd, add=True)

            @pl.when(j0 + 2 < CPP)
            def _():
                pltpu.async_copy(feat_hbm.at[src_v.at[j0 + 2]], rows0_v, sem0)

            pltpu.make_async_copy(feat_hbm.at[src_v.at[j0 + 1]], rows1_v,
                                  sem1).wait()
            pltpu.sync_copy(rows1_v, agg_sh.at[dst_v.at[j0 + 1]], add=True)
            pltpu.async_copy(ones_v, deg_sh.at[dst_v.at[j0 + 1]], semd,
                             add=True)
            return carry

        lax.fori_loop(0, CPP // 2, body, 0)

    def drain(j, carry):
        pltpu.make_async_copy(ones_hbm, ones_v, semd).wait()
        return carry

    lax.fori_loop(0, CH, drain, 0)
    plsc.subcore_barrier()
    pltpu.sync_copy(agg_sh.at[pl.ds(s * RPS, RPS)],
                    agg_hbm.at[c, pl.ds(s * RPS, RPS)])
    pltpu.sync_copy(deg_sh.at[pl.ds(s * RPS, RPS)],
                    deg_hbm.at[c, pl.ds(s * RPS, RPS)])


def _make_sc_agg():
    mesh = plsc.VectorSubcoreMesh(core_axis_name="c", subcore_axis_name="s")
    return pl.kernel(
        _sc_agg_body,
        out_type=(
            jax.ShapeDtypeStruct((NCORES, NP, D), jnp.bfloat16),
            jax.ShapeDtypeStruct((NCORES, NP, DG), jnp.float32),
        ),
        mesh=mesh,
        scratch_types=[
            pltpu.VMEM((CPP, K), jnp.int32),
            pltpu.VMEM((CPP, K), jnp.int32),
            pltpu.VMEM((K, DG), jnp.float32),
            pltpu.VMEM((K, D), jnp.bfloat16),
            pltpu.VMEM((K, D), jnp.bfloat16),
            pltpu.VMEM_SHARED((NP, D), jnp.bfloat16),
            pltpu.VMEM_SHARED((NP, DG), jnp.float32),
            pltpu.SemaphoreType.DMA,
            pltpu.SemaphoreType.DMA,
            pltpu.SemaphoreType.DMA,
            pltpu.SemaphoreType.DMA,
        ],
        compiler_params=pltpu.CompilerParams(use_tc_tiling_on_sc=False),
    )


def _conv_stats_body(a0_ref, a1_ref, d0_ref, d1_ref, fr_ref, wl_ref,
                     o_ref, st_ref, acc_ref):
    i = pl.program_id(0)

    @pl.when(i == 0)
    def _():
        acc_ref[...] = jnp.zeros_like(acc_ref)

    a = (a0_ref[0].astype(jnp.float32) + a1_ref[0].astype(jnp.float32))
    deg = jnp.sum(d0_ref[0] + d1_ref[0], axis=1, keepdims=True) * (1.0 / DG)
    mean = a / jnp.maximum(deg, 1.0)
    o = jnp.dot(mean, wl_ref[...], preferred_element_type=jnp.float32)
    o = o + fr_ref[...]
    o_ref[...] = o
    acc_ref[0:1, :] += jnp.sum(o, axis=0, keepdims=True)
    acc_ref[1:2, :] += jnp.sum(o * o, axis=0, keepdims=True)

    @pl.when(i == NB - 1)
    def _():
        st_ref[...] = acc_ref[...]


def _bn_body(o_ref, st_ref, g_ref, be_ref, out_ref):
    m = st_ref[0:1, :] / N
    v = st_ref[1:2, :] / N - m * m
    inv = g_ref[...] * lax.rsqrt(v + EPS)
    out_ref[...] = (o_ref[...] - m) * inv + be_ref[...]


def kernel(x, edge_index, W1, b1, g1, beta1, Wl, bl, Wr, g2, beta2):
    ar = jnp.arange(EPAD, dtype=jnp.int32)
    pad = jnp.stack([ar * 41 % N, N + (ar % (NP - N))])
    er = jnp.concatenate([edge_index, pad], axis=1)
    er = er.reshape(2, NTILES, CH, K)

    feat, feat_bf = pl.pallas_call(
        _lin_bn_relu_body,
        grid=(2 * NB,),
        in_specs=[
            pl.BlockSpec((BM, D), lambda i: (jnp.where(i < NB, i, 0), 0)),
            pl.BlockSpec((D, H), lambda i: (0, 0)),
            pl.BlockSpec((1, H), lambda i: (0, 0)),
            pl.BlockSpec((1, H), lambda i: (0, 0)),
            pl.BlockSpec((1, H), lambda i: (0, 0)),
        ],
        out_specs=[
            pl.BlockSpec((BM, H),
                         lambda i: (jnp.where(i < NB, 0, i - NB), 0)),
            pl.BlockSpec((BM, H),
                         lambda i: (jnp.where(i < NB, 0, i - NB), 0)),
        ],
        out_shape=[
            jax.ShapeDtypeStruct((N, H), jnp.float32),
            jax.ShapeDtypeStruct((N, H), jnp.bfloat16),
        ],
        scratch_shapes=[
            pltpu.VMEM((N, H), jnp.float32),
            pltpu.VMEM((8, H), jnp.float32),
        ],
    )(x, W1, b1.reshape(1, H), g1.reshape(1, H), beta1.reshape(1, H))

    ones = jnp.ones((K, DG), jnp.float32)
    zagg = jnp.zeros((RPS, D), jnp.bfloat16)
    zdeg = jnp.zeros((RPS, DG), jnp.float32)
    aggp, degp = _make_sc_agg()(feat_bf, er, ones, zagg, zdeg)

    fr = pl.pallas_call(
        _fr_body,
        grid=(NB,),
        in_specs=[
            pl.BlockSpec((BM, H), lambda i: (i, 0)),
            pl.BlockSpec((H, H), lambda i: (0, 0)),
            pl.BlockSpec((1, H), lambda i: (0, 0)),
        ],
        out_specs=pl.BlockSpec((BM, H), lambda i: (i, 0)),
        out_shape=jax.ShapeDtypeStruct((N, H), jnp.float32),
    )(feat, Wr, bl.reshape(1, H))

    out_raw, st2 = pl.pallas_call(
        _conv_stats_body,
        grid=(NB,),
        in_specs=[
            pl.BlockSpec((1, BM, D), lambda i: (0, i, 0)),
            pl.BlockSpec((1, BM, D), lambda i: (1, i, 0)),
            pl.BlockSpec((1, BM, DG), lambda i: (0, i, 0)),
            pl.BlockSpec((1, BM, DG), lambda i: (1, i, 0)),
            pl.BlockSpec((BM, H), lambda i: (i, 0)),
            pl.BlockSpec((H, H), lambda i: (0, 0)),
        ],
        out_specs=[
            pl.BlockSpec((BM, H), lambda i: (i, 0)),
            pl.BlockSpec((8, H), lambda i: (0, 0)),
        ],
        out_shape=[
            jax.ShapeDtypeStruct((N, H), jnp.float32),
            jax.ShapeDtypeStruct((8, H), jnp.float32),
        ],
        scratch_shapes=[pltpu.VMEM((8, H), jnp.float32)],
    )(aggp, aggp, degp, degp, fr, Wl)

    out_feat = pl.pallas_call(
        _bn_body,
        grid=(NB,),
        in_specs=[
            pl.BlockSpec((BM, H), lambda i: (i, 0)),
            pl.BlockSpec((8, H), lambda i: (0, 0)),
            pl.BlockSpec((1, H), lambda i: (0, 0)),
            pl.BlockSpec((1, H), lambda i: (0, 0)),
        ],
        out_specs=pl.BlockSpec((BM, H), lambda i: (i, 0)),
        out_shape=jax.ShapeDtypeStruct((N, H), jnp.float32),
    )(out_raw, st2, g2.reshape(1, H), beta2.reshape(1, H))

    return (feat, out_feat)
